# Initial kernel scaffold; baseline (speedup 1.0000x reference)
import jax
import jax.numpy as jnp
from jax.experimental import pallas as pl


def kernel(edge_index, features, W1, b1, W2, b2, W3, b3, gamma, beta, Wm1, bm1, Wm2, bm2):
    raise NotImplementedError("write your pallas kernel here")


# speedup vs baseline: 2.7194x; 2.7194x over previous
import functools

import jax
import jax.numpy as jnp
from jax import lax
from jax.experimental import pallas as pl
from jax.experimental.pallas import tpu as pltpu
from jax.experimental.pallas import tpu_sc as plsc

_N = 10000
_E = 160000
_DIN = 256
_DH = 512
_DOUT = 256
_EPS_BN = 1e-5

_NC = 2
_NS = 16
_DC = 128
_BATCH = 128
_EPB = 10240
_NBATCH = _EPB // _BATCH
_NPAD = 240
_NA = _N + _NPAD
_ZR = 80
_WR = _NA // _NS

_R = 1024


def _sc_mesh():
    return plsc.VectorSubcoreMesh(core_axis_name="c", subcore_axis_name="s")


@functools.partial(jax.jit, static_argnames=("n_chunks",))
def _sc_segsum(table, sidx, didx, n_chunks):
    cpc = n_chunks // _NC

    @functools.partial(
        pl.kernel,
        mesh=_sc_mesh(),
        out_type=jax.ShapeDtypeStruct((n_chunks, _NA, _DC), jnp.float32),
        scratch_types=[
            pltpu.VMEM((_NBATCH, _BATCH), jnp.int32),
            pltpu.VMEM((_NBATCH, _BATCH), jnp.int32),
            pltpu.VMEM((_BATCH, _DC), jnp.float32),
            pltpu.VMEM((_ZR, _DC), jnp.float32),
            pltpu.VMEM_SHARED((_NA, _DC), jnp.float32),
            pltpu.SemaphoreType.DMA,
        ],
    )
    def k(table_h, sidx_h, didx_h, zeros_h, out_h, sv, dv, rows, zbuf, acc,
          sem):
        core = lax.axis_index("c")
        sub = lax.axis_index("s")
        pltpu.sync_copy(zeros_h, zbuf)
        pltpu.sync_copy(didx_h.at[sub], dv)
        for j in range(cpc):
            c = core * cpc + j
            @pl.loop(0, 640 // _ZR)
            def _(z):
                pltpu.sync_copy(
                    zbuf, acc.at[pl.ds(sub * 640 + z * _ZR, _ZR)])
            pltpu.sync_copy(sidx_h.at[c, sub], sv)
            plsc.subcore_barrier()

            @pl.loop(0, _NBATCH)
            def _(b):
                pltpu.async_copy(table_h.at[sv.at[b]], rows, sem).wait()
                pltpu.sync_copy(rows, acc.at[dv.at[b]], add=True)

            plsc.subcore_barrier()
            pltpu.sync_copy(acc.at[pl.ds(sub * _WR, _WR)],
                            out_h.at[c, pl.ds(sub * _WR, _WR)])
            plsc.subcore_barrier()

    return k(table, sidx, didx, jnp.zeros((_ZR, _DC), jnp.float32))


@jax.jit
def _sc_degree(didx):
    half = _NBATCH // 2

    @functools.partial(
        pl.kernel,
        mesh=_sc_mesh(),
        out_type=jax.ShapeDtypeStruct((_NC, _NA, _DC), jnp.float32),
        scratch_types=[
            pltpu.VMEM((half, _BATCH), jnp.int32),
            pltpu.VMEM((_BATCH, _DC), jnp.float32),
            pltpu.VMEM((_ZR, _DC), jnp.float32),
            pltpu.VMEM_SHARED((_NA, _DC), jnp.float32),
            pltpu.SemaphoreType.DMA,
        ],
    )
    def k(didx_h, ones_h, zeros_h, out_h, dv, ones, zbuf, acc, sem):
        core = lax.axis_index("c")
        sub = lax.axis_index("s")
        pltpu.sync_copy(zeros_h, zbuf)
        pltpu.sync_copy(ones_h, ones)

        @pl.loop(0, 640 // _ZR)
        def _(z):
            pltpu.sync_copy(zbuf, acc.at[pl.ds(sub * 640 + z * _ZR, _ZR)])
        pltpu.sync_copy(didx_h.at[sub, core], dv)
        plsc.subcore_barrier()

        @pl.loop(0, half)
        def _(b):
            pltpu.sync_copy(ones, acc.at[dv.at[b]], add=True)

        plsc.subcore_barrier()
        pltpu.sync_copy(acc.at[pl.ds(sub * 640, 640)],
                        out_h.at[core, pl.ds(sub * 640, 640)])

    return k(didx, jnp.ones((_BATCH, _DC), jnp.float32),
             jnp.zeros((_ZR, _DC), jnp.float32))


def _rowscale_body(x_ref, s_ref, o_ref):
    o_ref[0] = x_ref[...] * s_ref[...]


@functools.partial(jax.jit, static_argnames=("n_chunks",))
def _tc_rowscale(x, scale, n_chunks):
    return pl.pallas_call(
        _rowscale_body,
        grid=(_NA // _R, n_chunks),
        in_specs=[
            pl.BlockSpec((_R, _DC), lambda i, c: (i, c)),
            pl.BlockSpec((_R, 1), lambda i, c: (i, 0)),
        ],
        out_specs=pl.BlockSpec((1, _R, _DC), lambda i, c: (c, i, 0)),
        out_shape=jax.ShapeDtypeStruct((n_chunks, _NA, _DC), jnp.float32),
    )(x, scale)


def _rowscale_chunked_body(x_ref, s_ref, o_ref):
    o_ref[0] = x_ref[0] * s_ref[...]


@jax.jit
def _tc_rowscale_chunked(xc, scale):
    c = xc.shape[0]
    return pl.pallas_call(
        _rowscale_chunked_body,
        grid=(_NA // _R, c),
        in_specs=[
            pl.BlockSpec((1, _R, _DC), lambda i, cc: (cc, i, 0)),
            pl.BlockSpec((_R, 1), lambda i, cc: (i, 0)),
        ],
        out_specs=pl.BlockSpec((1, _R, _DC), lambda i, cc: (cc, i, 0)),
        out_shape=jax.ShapeDtypeStruct((c, _NA, _DC), jnp.float32),
    )(xc, scale)


def _conv1_body(x0_ref, a0_ref, a1_ref, dinv_ref, wa_ref, wb_ref, wc_ref,
                b_ref, gs_ref, beta_ref, x1_ref, g_ref):
    dinv = dinv_ref[...]
    z = jnp.dot(x0_ref[...], wa_ref[...],
                preferred_element_type=jnp.float32)
    for c in range(2):
        z += jnp.dot(a0_ref[c] * dinv, wb_ref[pl.ds(c * _DC, _DC), :],
                     preferred_element_type=jnp.float32)
        z += jnp.dot(a1_ref[c] * dinv, wc_ref[pl.ds(c * _DC, _DC), :],
                     preferred_element_type=jnp.float32)
    x1 = jnp.maximum(z + b_ref[...], 0.0) * gs_ref[...] + beta_ref[...]
    x1_ref[...] = x1
    for c in range(4):
        g_ref[c] = x1[:, c * _DC:(c + 1) * _DC] * dinv


@jax.jit
def _tc_conv1(x0, a0c, a1c, dinv, wa, wb, wc, b, gs, beta):
    return pl.pallas_call(
        _conv1_body,
        grid=(_NA // _R,),
        in_specs=[
            pl.BlockSpec((_R, _DIN), lambda i: (i, 0)),
            pl.BlockSpec((2, _R, _DC), lambda i: (0, i, 0)),
            pl.BlockSpec((2, _R, _DC), lambda i: (0, i, 0)),
            pl.BlockSpec((_R, 1), lambda i: (i, 0)),
            pl.BlockSpec((_DIN, _DH), lambda i: (0, 0)),
            pl.BlockSpec((_DIN, _DH), lambda i: (0, 0)),
            pl.BlockSpec((_DIN, _DH), lambda i: (0, 0)),
            pl.BlockSpec((1, _DH), lambda i: (0, 0)),
            pl.BlockSpec((1, _DH), lambda i: (0, 0)),
            pl.BlockSpec((1, _DH), lambda i: (0, 0)),
        ],
        out_specs=[
            pl.BlockSpec((_R, _DH), lambda i: (i, 0)),
            pl.BlockSpec((4, _R, _DC), lambda i: (0, i, 0)),
        ],
        out_shape=[
            jax.ShapeDtypeStruct((_NA, _DH), jnp.float32),
            jax.ShapeDtypeStruct((4, _NA, _DC), jnp.float32),
        ],
    )(x0, a0c, a1c, dinv, wa, wb, wc, b, gs, beta)


def _conv3_mlp_body(x1_ref, a0_ref, a1_ref, dinv_ref, wa_ref, wb_ref, wc_ref,
                    b_ref, wm1_ref, bm1_ref, wm2_ref, bm2_ref, o_ref):
    dinv = dinv_ref[...]
    x1 = x1_ref[...]
    z = jnp.dot(x1, wa_ref[...], preferred_element_type=jnp.float32)
    for c in range(4):
        z += jnp.dot(a0_ref[c] * dinv, wb_ref[pl.ds(c * _DC, _DC), :],
                     preferred_element_type=jnp.float32)
        z += jnp.dot(a1_ref[c] * dinv, wc_ref[pl.ds(c * _DC, _DC), :],
                     preferred_element_type=jnp.float32)
    x = jnp.maximum(z + b_ref[...], 0.0) + x1
    h = jnp.maximum(jnp.dot(x, wm1_ref[...],
                            preferred_element_type=jnp.float32)
                    + bm1_ref[...], 0.0)
    o_ref[...] = jnp.dot(h, wm2_ref[...],
                         preferred_element_type=jnp.float32) + bm2_ref[...]


@jax.jit
def _tc_conv3_mlp(x1, a0c, a1c, dinv, wa, wb, wc, b, wm1, bm1, wm2, bm2):
    return pl.pallas_call(
        _conv3_mlp_body,
        grid=(_NA // _R,),
        in_specs=[
            pl.BlockSpec((_R, _DH), lambda i: (i, 0)),
            pl.BlockSpec((4, _R, _DC), lambda i: (0, i, 0)),
            pl.BlockSpec((4, _R, _DC), lambda i: (0, i, 0)),
            pl.BlockSpec((_R, 1), lambda i: (i, 0)),
            pl.BlockSpec((_DH, _DH), lambda i: (0, 0)),
            pl.BlockSpec((_DH, _DH), lambda i: (0, 0)),
            pl.BlockSpec((_DH, _DH), lambda i: (0, 0)),
            pl.BlockSpec((1, _DH), lambda i: (0, 0)),
            pl.BlockSpec((_DH, _DH), lambda i: (0, 0)),
            pl.BlockSpec((1, _DH), lambda i: (0, 0)),
            pl.BlockSpec((_DH, _DOUT), lambda i: (0, 0)),
            pl.BlockSpec((1, _DOUT), lambda i: (0, 0)),
        ],
        out_specs=pl.BlockSpec((_R, _DOUT), lambda i: (i, 0)),
        out_shape=jax.ShapeDtypeStruct((_NA, _DOUT), jnp.float32),
    )(x1, a0c, a1c, dinv, wa, wb, wc, b, wm1, bm1, wm2, bm2)


def _prep_indices(edge_index):
    src = edge_index[0]
    dst = edge_index[1]
    pad_dst = _N + (jnp.arange(_EPB - _E // _NS, dtype=jnp.int32) % _NPAD)
    srcp = jnp.concatenate(
        [src.reshape(_NS, _E // _NS),
         jnp.zeros((_NS, _EPB - _E // _NS), jnp.int32)], axis=1)
    dstp = jnp.concatenate(
        [dst.reshape(_NS, _E // _NS),
         jnp.broadcast_to(pad_dst, (_NS, _EPB - _E // _NS))], axis=1)
    shift2 = (jnp.arange(2, dtype=jnp.int32) * _NA)[:, None, None]
    shift4 = (jnp.arange(4, dtype=jnp.int32) * _NA)[:, None, None]
    sidx2 = (srcp[None] + shift2).reshape(2, _NS, _NBATCH, _BATCH)
    sidx4 = (srcp[None] + shift4).reshape(4, _NS, _NBATCH, _BATCH)
    didx = dstp.reshape(_NS, _NBATCH, _BATCH)
    didx_deg = dstp.reshape(_NS, 2, _NBATCH // 2, _BATCH)
    return sidx2, sidx4, didx, didx_deg


def kernel(edge_index, features, W1, b1, W2, b2, W3, b3, gamma, beta,
           Wm1, bm1, Wm2, bm2):
    del W2, b2

    sidx2, sidx4, didx, didx_deg = _prep_indices(edge_index)

    degp = _sc_degree(didx_deg)
    deg = degp[0, :, 0] + degp[1, :, 0]
    dinv = lax.rsqrt(jnp.maximum(deg, 1.0))[:, None]
    dinv2 = (1.0 / jnp.maximum(deg, 1.0))[:, None]
    feats = jnp.concatenate(
        [features, jnp.zeros((_NA - _N, _DIN), jnp.float32)], axis=0)

    w1a = W1[:_DIN] - W1[2 * _DIN:]
    w1b = -W1[_DIN:2 * _DIN]
    w1c = 2.0 * W1[2 * _DIN:]
    w3a = W3[:_DH] - W3[2 * _DH:]
    w3b = -W3[_DH:2 * _DH]
    w3c = 2.0 * W3[2 * _DH:]
    gs = (gamma / jnp.sqrt(1.0 + _EPS_BN))[None, :]

    g0 = _tc_rowscale(feats, dinv, 2)
    a0 = _sc_segsum(g0.reshape(2 * _NA, _DC), sidx2, didx, 2)
    g1 = _tc_rowscale_chunked(a0, dinv2)
    a1 = _sc_segsum(g1.reshape(2 * _NA, _DC), sidx2, didx, 2)
    x1, g0p = _tc_conv1(feats, a0, a1, dinv, w1a, w1b, w1c,
                        b1[None, :], gs, beta[None, :])

    a0p = _sc_segsum(g0p.reshape(4 * _NA, _DC), sidx4, didx, 4)
    g1p = _tc_rowscale_chunked(a0p, dinv2)
    a1p = _sc_segsum(g1p.reshape(4 * _NA, _DC), sidx4, didx, 4)
    out = _tc_conv3_mlp(x1, a0p, a1p, dinv, w3a, w3b, w3c, b3[None, :],
                        Wm1, bm1[None, :], Wm2, bm2[None, :])
    return out[:_N]


# speedup vs baseline: 3.0693x; 1.1287x over previous
import functools

import jax
import jax.numpy as jnp
from jax import lax
from jax.experimental import pallas as pl
from jax.experimental.pallas import tpu as pltpu
from jax.experimental.pallas import tpu_sc as plsc

_N = 10000
_E = 160000
_DIN = 256
_DH = 512
_DOUT = 256
_EPS_BN = 1e-5

_NC = 2
_NS = 16
_DC = 128
_BATCH = 128
_EPB = 10240
_NBATCH = _EPB // _BATCH
_NPAD = 240
_NA = _N + _NPAD
_ZR = 16
_WR = _NA // _NS

_R = 1024


def _sc_mesh():
    return plsc.VectorSubcoreMesh(core_axis_name="c", subcore_axis_name="s")


@functools.partial(jax.jit, static_argnames=("n_chunks",))
def _sc_segsum(table, sidx, didx, n_chunks):
    cpc = n_chunks // _NC

    @functools.partial(
        pl.kernel,
        mesh=_sc_mesh(),
        out_type=jax.ShapeDtypeStruct((n_chunks, _NA, _DC), jnp.float32),
        scratch_types=[
            pltpu.VMEM((_NBATCH // 2, _BATCH), jnp.int32),
            pltpu.VMEM((_NBATCH // 2, _BATCH), jnp.int32),
            pltpu.VMEM((_BATCH, _DC), jnp.float32),
            pltpu.VMEM((_BATCH, _DC), jnp.float32),
            pltpu.VMEM((_ZR, _DC), jnp.float32),
            pltpu.VMEM_SHARED((_NA, _DC), jnp.float32),
            pltpu.SemaphoreType.DMA,
            pltpu.SemaphoreType.DMA,
        ],
    )
    def k(table_h, sidx_h, didx_h, zeros_h, out_h, sv, dv, r0, r1,
          zbuf, acc, gsem, ssem):
        core = lax.axis_index("c")
        sub = lax.axis_index("s")
        bufs = (r0, r1)
        nb2 = _NBATCH // 2
        pltpu.sync_copy(zeros_h, zbuf)
        for j in range(cpc):
            c = core * cpc + j
            @pl.loop(0, 640 // _ZR)
            def _(z):
                pltpu.sync_copy(
                    zbuf, acc.at[pl.ds(sub * 640 + z * _ZR, _ZR)])
            plsc.subcore_barrier()

            for half in range(2):
                pltpu.sync_copy(sidx_h.at[c, sub, pl.ds(half * nb2, nb2)], sv)
                pltpu.sync_copy(didx_h.at[sub, pl.ds(half * nb2, nb2)], dv)
                pltpu.async_copy(table_h.at[sv.at[0]], r0, gsem)

                @pl.loop(0, nb2, step=2)
                def _(b):
                    for j in range(2):
                        bb = b + j
                        cur = bufs[j]
                        nxt = bufs[1 - j]
                        pltpu.make_async_copy(
                            table_h.at[sv.at[bb]], cur, gsem).wait()

                        @pl.when(bb + 1 < nb2)
                        def _():
                            pltpu.async_copy(
                                table_h.at[sv.at[bb + 1]], nxt, gsem)

                        pltpu.async_copy(cur, acc.at[dv.at[bb]], ssem,
                                         add=True)
                        pltpu.make_async_copy(
                            cur, acc.at[dv.at[bb]], ssem).wait()

            plsc.subcore_barrier()
            pltpu.sync_copy(acc.at[pl.ds(sub * _WR, _WR)],
                            out_h.at[c, pl.ds(sub * _WR, _WR)])
            plsc.subcore_barrier()

    return k(table, sidx, didx, jnp.zeros((_ZR, _DC), jnp.float32))


@jax.jit
def _sc_degree(didx):
    half = _NBATCH // 2

    @functools.partial(
        pl.kernel,
        mesh=_sc_mesh(),
        out_type=jax.ShapeDtypeStruct((_NC, _NA, _DC), jnp.float32),
        scratch_types=[
            pltpu.VMEM((half, _BATCH), jnp.int32),
            pltpu.VMEM((_BATCH, _DC), jnp.float32),
            pltpu.VMEM((_ZR, _DC), jnp.float32),
            pltpu.VMEM_SHARED((_NA, _DC), jnp.float32),
            pltpu.SemaphoreType.DMA,
        ],
    )
    def k(didx_h, ones_h, zeros_h, out_h, dv, ones, zbuf, acc, sem):
        core = lax.axis_index("c")
        sub = lax.axis_index("s")
        pltpu.sync_copy(zeros_h, zbuf)
        pltpu.sync_copy(ones_h, ones)

        @pl.loop(0, 640 // _ZR)
        def _(z):
            pltpu.sync_copy(zbuf, acc.at[pl.ds(sub * 640 + z * _ZR, _ZR)])
        pltpu.sync_copy(didx_h.at[sub, core], dv)
        plsc.subcore_barrier()

        @pl.loop(0, half)
        def _(b):
            pltpu.sync_copy(ones, acc.at[dv.at[b]], add=True)

        plsc.subcore_barrier()
        pltpu.sync_copy(acc.at[pl.ds(sub * 640, 640)],
                        out_h.at[core, pl.ds(sub * 640, 640)])

    return k(didx, jnp.ones((_BATCH, _DC), jnp.float32),
             jnp.zeros((_ZR, _DC), jnp.float32))


def _rowscale_body(x_ref, s_ref, o_ref):
    o_ref[0] = x_ref[...] * s_ref[...]


@functools.partial(jax.jit, static_argnames=("n_chunks",))
def _tc_rowscale(x, scale, n_chunks):
    return pl.pallas_call(
        _rowscale_body,
        grid=(_NA // _R, n_chunks),
        in_specs=[
            pl.BlockSpec((_R, _DC), lambda i, c: (i, c)),
            pl.BlockSpec((_R, 1), lambda i, c: (i, 0)),
        ],
        out_specs=pl.BlockSpec((1, _R, _DC), lambda i, c: (c, i, 0)),
        out_shape=jax.ShapeDtypeStruct((n_chunks, _NA, _DC), jnp.float32),
    )(x, scale)


def _rowscale_chunked_body(x_ref, s_ref, o_ref):
    o_ref[0] = x_ref[0] * s_ref[...]


@jax.jit
def _tc_rowscale_chunked(xc, scale):
    c = xc.shape[0]
    return pl.pallas_call(
        _rowscale_chunked_body,
        grid=(_NA // _R, c),
        in_specs=[
            pl.BlockSpec((1, _R, _DC), lambda i, cc: (cc, i, 0)),
            pl.BlockSpec((_R, 1), lambda i, cc: (i, 0)),
        ],
        out_specs=pl.BlockSpec((1, _R, _DC), lambda i, cc: (cc, i, 0)),
        out_shape=jax.ShapeDtypeStruct((c, _NA, _DC), jnp.float32),
    )(xc, scale)


def _conv1_body(x0_ref, a0_ref, a1_ref, dinv_ref, wa_ref, wb_ref, wc_ref,
                b_ref, gs_ref, beta_ref, x1_ref, g_ref):
    dinv = dinv_ref[...]
    z = jnp.dot(x0_ref[...], wa_ref[...],
                preferred_element_type=jnp.float32)
    for c in range(2):
        z += jnp.dot(a0_ref[c] * dinv, wb_ref[pl.ds(c * _DC, _DC), :],
                     preferred_element_type=jnp.float32)
        z += jnp.dot(a1_ref[c] * dinv, wc_ref[pl.ds(c * _DC, _DC), :],
                     preferred_element_type=jnp.float32)
    x1 = jnp.maximum(z + b_ref[...], 0.0) * gs_ref[...] + beta_ref[...]
    x1_ref[...] = x1
    for c in range(4):
        g_ref[c] = x1[:, c * _DC:(c + 1) * _DC] * dinv


@jax.jit
def _tc_conv1(x0, a0c, a1c, dinv, wa, wb, wc, b, gs, beta):
    return pl.pallas_call(
        _conv1_body,
        grid=(_NA // _R,),
        in_specs=[
            pl.BlockSpec((_R, _DIN), lambda i: (i, 0)),
            pl.BlockSpec((2, _R, _DC), lambda i: (0, i, 0)),
            pl.BlockSpec((2, _R, _DC), lambda i: (0, i, 0)),
            pl.BlockSpec((_R, 1), lambda i: (i, 0)),
            pl.BlockSpec((_DIN, _DH), lambda i: (0, 0)),
            pl.BlockSpec((_DIN, _DH), lambda i: (0, 0)),
            pl.BlockSpec((_DIN, _DH), lambda i: (0, 0)),
            pl.BlockSpec((1, _DH), lambda i: (0, 0)),
            pl.BlockSpec((1, _DH), lambda i: (0, 0)),
            pl.BlockSpec((1, _DH), lambda i: (0, 0)),
        ],
        out_specs=[
            pl.BlockSpec((_R, _DH), lambda i: (i, 0)),
            pl.BlockSpec((4, _R, _DC), lambda i: (0, i, 0)),
        ],
        out_shape=[
            jax.ShapeDtypeStruct((_NA, _DH), jnp.float32),
            jax.ShapeDtypeStruct((4, _NA, _DC), jnp.float32),
        ],
    )(x0, a0c, a1c, dinv, wa, wb, wc, b, gs, beta)


def _conv3_mlp_body(x1_ref, a0_ref, a1_ref, dinv_ref, wa_ref, wb_ref, wc_ref,
                    b_ref, wm1_ref, bm1_ref, wm2_ref, bm2_ref, o_ref):
    dinv = dinv_ref[...]
    x1 = x1_ref[...]
    z = jnp.dot(x1, wa_ref[...], preferred_element_type=jnp.float32)
    for c in range(4):
        z += jnp.dot(a0_ref[c] * dinv, wb_ref[pl.ds(c * _DC, _DC), :],
                     preferred_element_type=jnp.float32)
        z += jnp.dot(a1_ref[c] * dinv, wc_ref[pl.ds(c * _DC, _DC), :],
                     preferred_element_type=jnp.float32)
    x = jnp.maximum(z + b_ref[...], 0.0) + x1
    h = jnp.maximum(jnp.dot(x, wm1_ref[...],
                            preferred_element_type=jnp.float32)
                    + bm1_ref[...], 0.0)
    o_ref[...] = jnp.dot(h, wm2_ref[...],
                         preferred_element_type=jnp.float32) + bm2_ref[...]


@jax.jit
def _tc_conv3_mlp(x1, a0c, a1c, dinv, wa, wb, wc, b, wm1, bm1, wm2, bm2):
    return pl.pallas_call(
        _conv3_mlp_body,
        grid=(_NA // _R,),
        in_specs=[
            pl.BlockSpec((_R, _DH), lambda i: (i, 0)),
            pl.BlockSpec((4, _R, _DC), lambda i: (0, i, 0)),
            pl.BlockSpec((4, _R, _DC), lambda i: (0, i, 0)),
            pl.BlockSpec((_R, 1), lambda i: (i, 0)),
            pl.BlockSpec((_DH, _DH), lambda i: (0, 0)),
            pl.BlockSpec((_DH, _DH), lambda i: (0, 0)),
            pl.BlockSpec((_DH, _DH), lambda i: (0, 0)),
            pl.BlockSpec((1, _DH), lambda i: (0, 0)),
            pl.BlockSpec((_DH, _DH), lambda i: (0, 0)),
            pl.BlockSpec((1, _DH), lambda i: (0, 0)),
            pl.BlockSpec((_DH, _DOUT), lambda i: (0, 0)),
            pl.BlockSpec((1, _DOUT), lambda i: (0, 0)),
        ],
        out_specs=pl.BlockSpec((_R, _DOUT), lambda i: (i, 0)),
        out_shape=jax.ShapeDtypeStruct((_NA, _DOUT), jnp.float32),
    )(x1, a0c, a1c, dinv, wa, wb, wc, b, wm1, bm1, wm2, bm2)


def _prep_indices(edge_index):
    src = edge_index[0]
    dst = edge_index[1]
    pad_dst = _N + (jnp.arange(_EPB - _E // _NS, dtype=jnp.int32) % _NPAD)
    srcp = jnp.concatenate(
        [src.reshape(_NS, _E // _NS),
         jnp.zeros((_NS, _EPB - _E // _NS), jnp.int32)], axis=1)
    dstp = jnp.concatenate(
        [dst.reshape(_NS, _E // _NS),
         jnp.broadcast_to(pad_dst, (_NS, _EPB - _E // _NS))], axis=1)
    shift2 = (jnp.arange(2, dtype=jnp.int32) * _NA)[:, None, None]
    shift4 = (jnp.arange(4, dtype=jnp.int32) * _NA)[:, None, None]
    sidx2 = (srcp[None] + shift2).reshape(2, _NS, _NBATCH, _BATCH)
    sidx4 = (srcp[None] + shift4).reshape(4, _NS, _NBATCH, _BATCH)
    didx = dstp.reshape(_NS, _NBATCH, _BATCH)
    didx_deg = dstp.reshape(_NS, 2, _NBATCH // 2, _BATCH)
    return sidx2, sidx4, didx, didx_deg


def kernel(edge_index, features, W1, b1, W2, b2, W3, b3, gamma, beta,
           Wm1, bm1, Wm2, bm2):
    del W2, b2

    sidx2, sidx4, didx, didx_deg = _prep_indices(edge_index)

    degp = _sc_degree(didx_deg)
    deg = degp[0, :, 0] + degp[1, :, 0]
    dinv = lax.rsqrt(jnp.maximum(deg, 1.0))[:, None]
    dinv2 = (1.0 / jnp.maximum(deg, 1.0))[:, None]
    feats = jnp.concatenate(
        [features, jnp.zeros((_NA - _N, _DIN), jnp.float32)], axis=0)

    w1a = W1[:_DIN] - W1[2 * _DIN:]
    w1b = -W1[_DIN:2 * _DIN]
    w1c = 2.0 * W1[2 * _DIN:]
    w3a = W3[:_DH] - W3[2 * _DH:]
    w3b = -W3[_DH:2 * _DH]
    w3c = 2.0 * W3[2 * _DH:]
    gs = (gamma / jnp.sqrt(1.0 + _EPS_BN))[None, :]

    g0 = _tc_rowscale(feats, dinv, 2)
    a0 = _sc_segsum(g0.reshape(2 * _NA, _DC), sidx2, didx, 2)
    g1 = _tc_rowscale_chunked(a0, dinv2)
    a1 = _sc_segsum(g1.reshape(2 * _NA, _DC), sidx2, didx, 2)
    x1, g0p = _tc_conv1(feats, a0, a1, dinv, w1a, w1b, w1c,
                        b1[None, :], gs, beta[None, :])

    a0p = _sc_segsum(g0p.reshape(4 * _NA, _DC), sidx4, didx, 4)
    g1p = _tc_rowscale_chunked(a0p, dinv2)
    a1p = _sc_segsum(g1p.reshape(4 * _NA, _DC), sidx4, didx, 4)
    out = _tc_conv3_mlp(x1, a0p, a1p, dinv, w3a, w3b, w3c, b3[None, :],
                        Wm1, bm1[None, :], Wm2, bm2[None, :])
    return out[:_N]
